# baseline (speedup 1.0000x reference)
import functools
import math

import jax
import jax.numpy as jnp
from jax import lax
from jax.experimental import pallas as pl
from jax.experimental.pallas import tpu as pltpu
from jax.experimental.pallas import tpu_sc as plsc

N = 10000
E = 160000
NW = 32
CH = 128
NCH = 40
PW = NCH * CH
E_PAD = NW * PW
NACC = 10240
TRASH = 10000
TPR = NACC // 16
BLK = 3200
BLKN = 1000
INV_SQRT_D = 1.0 / math.sqrt(128.0)

_mesh = plsc.VectorSubcoreMesh(core_axis_name="c", subcore_axis_name="s")


@functools.partial(
    pl.kernel,
    mesh=_mesh,
    out_type=jax.ShapeDtypeStruct((E_PAD, 128), jnp.float32),
    scratch_types=[
        pltpu.VMEM((NCH, CH), jnp.int32),
        pltpu.VMEM((2 * CH, 128), jnp.float32),
        pltpu.SemaphoreType.DMA,
        pltpu.SemaphoreType.DMA,
    ],
)
def _gather_q(q_hbm, idx_hbm, qg_hbm, idx_v, rows_v, gsem, wsem):
    wid = lax.axis_index("s") * 2 + lax.axis_index("c")
    pltpu.sync_copy(idx_hbm.at[wid], idx_v)
    base = wid * PW

    pltpu.async_copy(q_hbm.at[idx_v.at[0]], rows_v.at[pl.ds(0, CH)], gsem)

    def body(j, carry):
        cur = rows_v.at[pl.ds((j % 2) * CH, CH)]
        nxt = rows_v.at[pl.ds(((j + 1) % 2) * CH, CH)]

        @pl.when(j >= 1)
        def _():
            pltpu.make_async_copy(
                q_hbm.at[pl.ds(0, CH)], nxt, wsem).wait()

        @pl.when(j + 1 < NCH)
        def _():
            pltpu.async_copy(q_hbm.at[idx_v.at[j + 1]], nxt, gsem)

        pltpu.make_async_copy(q_hbm.at[pl.ds(0, CH)], cur, gsem).wait()
        pltpu.async_copy(cur, qg_hbm.at[pl.ds(base + j * CH, CH)], wsem)
        return carry

    lax.fori_loop(0, NCH, body, 0)
    pltpu.make_async_copy(
        q_hbm.at[pl.ds(0, CH)], rows_v.at[pl.ds(0, CH)], wsem).wait()


def _head_mat(nf, group):
    f = lax.broadcasted_iota(jnp.int32, (nf, 8), 0)
    h = lax.broadcasted_iota(jnp.int32, (nf, 8), 1)
    return jnp.where(f // group == h, 1.0, 0.0).astype(jnp.float32)


def _edge_body(k0_r, k1_r, v0_r, v1_r, qg_r, w_r, ex_r):
    qg = qg_r[...]
    p0 = k0_r[...] * qg[:, :32]
    p1 = k1_r[...] * qg[:, 32:]
    g0 = _head_mat(32, 4)
    g1 = _head_mat(96, 12)
    e8 = (jnp.dot(p0, g0, preferred_element_type=jnp.float32)
          + jnp.dot(p1, g1, preferred_element_type=jnp.float32)) * INV_SQRT_D
    ex = jnp.exp(e8)
    eh0 = jnp.dot(ex, g0.T, preferred_element_type=jnp.float32)
    eh1 = jnp.dot(ex, g1.T, preferred_element_type=jnp.float32)
    w_r[...] = jnp.concatenate([eh0 * v0_r[...], eh1 * v1_r[...]], axis=1)
    ex_r[...] = jnp.concatenate([eh0, eh1], axis=1)


def _edge_math(k0, k1, v0, v1, qg):
    nblk = E // BLK
    return pl.pallas_call(
        _edge_body,
        grid=(nblk,),
        in_specs=[
            pl.BlockSpec((BLK, 32), lambda i: (i, 0)),
            pl.BlockSpec((BLK, 96), lambda i: (i, 0)),
            pl.BlockSpec((BLK, 32), lambda i: (i, 0)),
            pl.BlockSpec((BLK, 96), lambda i: (i, 0)),
            pl.BlockSpec((BLK, 128), lambda i: (i, 0)),
        ],
        out_specs=[
            pl.BlockSpec((BLK, 128), lambda i: (i, 0)),
            pl.BlockSpec((BLK, 128), lambda i: (i, 0)),
        ],
        out_shape=[
            jax.ShapeDtypeStruct((E_PAD, 128), jnp.float32),
            jax.ShapeDtypeStruct((E_PAD, 128), jnp.float32),
        ],
    )(k0, k1, v0, v1, qg)


@functools.partial(
    pl.kernel,
    mesh=_mesh,
    out_type=jax.ShapeDtypeStruct((2, NACC, 128), jnp.float32),
    scratch_types=[
        pltpu.VMEM((NCH, CH), jnp.int32),
        pltpu.VMEM((2 * CH, 128), jnp.float32),
        pltpu.VMEM((16, 128), jnp.float32),
        pltpu.VMEM_SHARED((NACC, 128), jnp.float32),
        pltpu.SemaphoreType.DMA,
        pltpu.SemaphoreType.DMA,
    ],
)
def _scatter_rows(x_hbm, idx_hbm, acc_hbm, idx_v, x_v, z_u, u_sh, gsem, ssem):
    c = lax.axis_index("c")
    sc = lax.axis_index("s")
    wid = sc * 2 + c
    zero16 = jnp.zeros((16,), jnp.float32)
    for r in range(16):
        for g in range(8):
            z_u[r, pl.ds(g * 16, 16)] = zero16
    row0 = sc * TPR

    def zbody(t, carry):
        pltpu.sync_copy(z_u, u_sh.at[pl.ds(row0 + t * 16, 16)])
        return carry

    lax.fori_loop(0, TPR // 16, zbody, 0)
    plsc.subcore_barrier()

    pltpu.sync_copy(idx_hbm.at[wid], idx_v)
    base = wid * PW

    pltpu.async_copy(x_hbm.at[pl.ds(base, CH)], x_v.at[pl.ds(0, CH)], gsem)

    def body(j, carry):
        cur = x_v.at[pl.ds((j % 2) * CH, CH)]
        nxt = x_v.at[pl.ds(((j + 1) % 2) * CH, CH)]

        @pl.when(j >= 1)
        def _():
            pltpu.make_async_copy(x_hbm.at[pl.ds(0, CH)], nxt, ssem).wait()

        @pl.when(j + 1 < NCH)
        def _():
            pltpu.async_copy(
                x_hbm.at[pl.ds(base + (j + 1) * CH, CH)], nxt, gsem)

        pltpu.make_async_copy(x_hbm.at[pl.ds(0, CH)], cur, gsem).wait()
        pltpu.async_copy(cur, u_sh.at[idx_v.at[j]], ssem, add=True)
        return carry

    lax.fori_loop(0, NCH, body, 0)
    pltpu.make_async_copy(
        x_hbm.at[pl.ds(0, CH)], x_v.at[pl.ds(0, CH)], ssem).wait()
    plsc.subcore_barrier()

    def ebody(t, carry):
        buf = x_v.at[pl.ds(0, CH)]
        pltpu.sync_copy(u_sh.at[pl.ds(row0 + t * CH, CH)], buf)
        pltpu.sync_copy(buf, acc_hbm.at[c, pl.ds(row0 + t * CH, CH)])
        return carry

    lax.fori_loop(0, TPR // CH, ebody, 0)


def _norm_body(up_r, rp_r, o0_r, o1_r):
    u = up_r[0] + up_r[1]
    rep = rp_r[0] + rp_r[1]
    out = u / (rep + 1e-9)
    o0_r[...] = out[:, :32]
    o1_r[...] = out[:, 32:]


def _normalize(up, rp):
    return pl.pallas_call(
        _norm_body,
        grid=(N // BLKN,),
        in_specs=[
            pl.BlockSpec((2, BLKN, 128), lambda i: (0, i, 0)),
            pl.BlockSpec((2, BLKN, 128), lambda i: (0, i, 0)),
        ],
        out_specs=[
            pl.BlockSpec((BLKN, 32), lambda i: (i, 0)),
            pl.BlockSpec((BLKN, 96), lambda i: (i, 0)),
        ],
        out_shape=[
            jax.ShapeDtypeStruct((N, 32), jnp.float32),
            jax.ShapeDtypeStruct((N, 96), jnp.float32),
        ],
    )(up, rp)


def kernel(edge_index, qry_0, qry_1, key_0, key_1, val_0, val_1):
    dst = edge_index[1].astype(jnp.int32)
    qcat = jnp.concatenate(
        [qry_0.reshape(N, 32), qry_1.reshape(N, 96)], axis=1)
    k0 = key_0.reshape(E, 32)
    k1 = key_1.reshape(E, 96)
    v0 = val_0.reshape(E, 32)
    v1 = val_1.reshape(E, 96)
    idx_g = jnp.concatenate(
        [dst, jnp.zeros((E_PAD - E,), jnp.int32)]).reshape(NW, NCH, CH)
    idx_s = jnp.concatenate(
        [dst, jnp.full((E_PAD - E,), TRASH, jnp.int32)]).reshape(NW, NCH, CH)

    qg = _gather_q(qcat, idx_g)
    w, exr = _edge_math(k0, k1, v0, v1, qg)
    up = _scatter_rows(w, idx_s)
    rp = _scatter_rows(exr, idx_s)
    o0, o1 = _normalize(up, rp)
    return o0.reshape(N, 32, 1), o1.reshape(N, 32, 3)
